# initial kernel scaffold (unmeasured)
import jax
import jax.numpy as jnp
from jax import lax
from jax.experimental import pallas as pl
from jax.experimental.pallas import tpu as pltpu


def kernel(
    x,
):
    def body(*refs):
        pass

    out_shape = jax.ShapeDtypeStruct(..., jnp.float32)
    return pl.pallas_call(body, out_shape=out_shape)(...)


# baseline (device time: 7000 ns/iter reference)
import jax
import jax.numpy as jnp
from jax import lax
from jax.experimental import pallas as pl
from jax.experimental.pallas import tpu as pltpu

N_DEV = 4
BIG = 1e9


def kernel(x):
    m_per, n = x.shape

    def body(x_ref, out_ref, mine_ref, comm_ref, send_sems, recv_sems):
        my_pos = lax.axis_index("i")

        barrier_sem = pltpu.get_barrier_semaphore()
        for d in range(1, N_DEV):
            pl.semaphore_signal(
                barrier_sem, inc=1,
                device_id=((my_pos + d) % N_DEV,),
                device_id_type=pl.DeviceIdType.MESH,
            )
        pl.semaphore_wait(barrier_sem, N_DEV - 1)

        vals = x_ref[:, :]
        mx = jnp.max(vals, axis=0, keepdims=True)
        row_ids = lax.broadcasted_iota(jnp.int32, (m_per, n), 0).astype(
            jnp.float32
        )
        loc_idx = jnp.min(
            jnp.where(vals == mx, row_ids, BIG), axis=0, keepdims=True
        )
        g_idx = loc_idx + my_pos.astype(jnp.float32) * m_per
        mine_ref[0:1, :] = mx
        mine_ref[1:2, :] = g_idx

        rdmas = []
        for d in range(1, N_DEV):
            rdma = pltpu.make_async_remote_copy(
                src_ref=mine_ref,
                dst_ref=comm_ref.at[N_DEV - 1 - d],
                send_sem=send_sems.at[d - 1],
                recv_sem=recv_sems.at[N_DEV - 1 - d],
                device_id=((my_pos + d) % N_DEV,),
                device_id_type=pl.DeviceIdType.MESH,
            )
            rdma.start()
            rdmas.append(rdma)

        for j in range(N_DEV - 1):
            recv = pltpu.make_async_remote_copy(
                src_ref=mine_ref,
                dst_ref=comm_ref.at[j],
                send_sem=send_sems.at[j],
                recv_sem=recv_sems.at[j],
                device_id=(my_pos,),
                device_id_type=pl.DeviceIdType.MESH,
            )
            recv.wait_recv()

        all_vals = jnp.concatenate(
            [mine_ref[0:1, :], comm_ref[:, 0, :]], axis=0
        )
        all_idx = jnp.concatenate(
            [mine_ref[1:2, :], comm_ref[:, 1, :]], axis=0
        )
        best = jnp.max(all_vals, axis=0, keepdims=True)
        best_idx = jnp.min(
            jnp.where(all_vals == best, all_idx, BIG), axis=0, keepdims=True
        )
        out_ref[0:1, :] = best
        out_ref[1:2, :] = best_idx

        for rdma in rdmas:
            rdma.wait_send()

    return pl.pallas_call(
        body,
        out_shape=jax.ShapeDtypeStruct((2, n), jnp.float32),
        in_specs=[pl.BlockSpec(memory_space=pltpu.VMEM)],
        out_specs=pl.BlockSpec(memory_space=pltpu.VMEM),
        scratch_shapes=[
            pltpu.VMEM((2, n), jnp.float32),
            pltpu.VMEM((N_DEV - 1, 2, n), jnp.float32),
            pltpu.SemaphoreType.DMA((N_DEV - 1,)),
            pltpu.SemaphoreType.DMA((N_DEV - 1,)),
        ],
        compiler_params=pltpu.CompilerParams(collective_id=0),
    )(x)


# device time: 6807 ns/iter; 1.0284x vs baseline; 1.0284x over previous
import jax
import jax.numpy as jnp
from jax import lax
from jax.experimental import pallas as pl
from jax.experimental.pallas import tpu as pltpu

N_DEV = 4
BIG = 1e9


def kernel(x):
    m_per, n = x.shape

    def body(x_ref, out_ref, mine_ref, comm_ref, send_sems, recv_sems):
        my_pos = lax.axis_index("i")

        barrier_sem = pltpu.get_barrier_semaphore()
        for d in range(1, N_DEV):
            pl.semaphore_signal(
                barrier_sem, inc=1,
                device_id=((my_pos + d) % N_DEV,),
                device_id_type=pl.DeviceIdType.MESH,
            )

        vals = x_ref[:, :]
        mx = jnp.max(vals, axis=0, keepdims=True)
        row_ids = lax.broadcasted_iota(jnp.int32, (m_per, n), 0).astype(
            jnp.float32
        )
        loc_idx = jnp.min(
            jnp.where(vals == mx, row_ids, BIG), axis=0, keepdims=True
        )
        g_idx = loc_idx + my_pos.astype(jnp.float32) * m_per
        mine_ref[0:1, :] = mx
        mine_ref[1:2, :] = g_idx

        pl.semaphore_wait(barrier_sem, N_DEV - 1)

        rdmas = []
        for d in range(1, N_DEV):
            rdma = pltpu.make_async_remote_copy(
                src_ref=mine_ref,
                dst_ref=comm_ref.at[N_DEV - 1 - d],
                send_sem=send_sems.at[d - 1],
                recv_sem=recv_sems.at[N_DEV - 1 - d],
                device_id=((my_pos + d) % N_DEV,),
                device_id_type=pl.DeviceIdType.MESH,
            )
            rdma.start()
            rdmas.append(rdma)

        for j in range(N_DEV - 1):
            recv = pltpu.make_async_remote_copy(
                src_ref=mine_ref,
                dst_ref=comm_ref.at[j],
                send_sem=send_sems.at[j],
                recv_sem=recv_sems.at[j],
                device_id=(my_pos,),
                device_id_type=pl.DeviceIdType.MESH,
            )
            recv.wait_recv()

        all_vals = jnp.concatenate(
            [mine_ref[0:1, :], comm_ref[:, 0, :]], axis=0
        )
        all_idx = jnp.concatenate(
            [mine_ref[1:2, :], comm_ref[:, 1, :]], axis=0
        )
        best = jnp.max(all_vals, axis=0, keepdims=True)
        best_idx = jnp.min(
            jnp.where(all_vals == best, all_idx, BIG), axis=0, keepdims=True
        )
        out_ref[0:1, :] = best
        out_ref[1:2, :] = best_idx

        for rdma in rdmas:
            rdma.wait_send()

    return pl.pallas_call(
        body,
        out_shape=jax.ShapeDtypeStruct((2, n), jnp.float32),
        in_specs=[pl.BlockSpec(memory_space=pltpu.VMEM)],
        out_specs=pl.BlockSpec(memory_space=pltpu.VMEM),
        scratch_shapes=[
            pltpu.VMEM((2, n), jnp.float32),
            pltpu.VMEM((N_DEV - 1, 2, n), jnp.float32),
            pltpu.SemaphoreType.DMA((N_DEV - 1,)),
            pltpu.SemaphoreType.DMA((N_DEV - 1,)),
        ],
        compiler_params=pltpu.CompilerParams(collective_id=0),
    )(x)
